# baseline (speedup 1.0000x reference)
import functools

import jax
import jax.numpy as jnp
from jax import lax
from jax.experimental import pallas as pl
from jax.experimental.pallas import tpu as pltpu
from jax.experimental.pallas import tpu_sc as plsc

NUM_LEVELS = 256
LANES = 16
NUM_CORES = 2
NUM_SUBCORES = 16
NUM_WORKERS = NUM_CORES * NUM_SUBCORES


def _sc_hist_body(data_hbm, out_hbm, rows_v, hist_v, cnt_v, sem0, sem1):
    n_rows, width = data_hbm.shape
    rows_per_w = n_rows // NUM_WORKERS
    half = rows_per_w // 2
    chunks = width // LANES

    wid = lax.axis_index("s") * NUM_CORES + lax.axis_index("c")
    base = wid * rows_per_w
    copy0 = pltpu.async_copy(
        data_hbm.at[pl.ds(base, half)], rows_v.at[pl.ds(0, half)], sem0)
    copy1 = pltpu.async_copy(
        data_hbm.at[pl.ds(base + half, half)],
        rows_v.at[pl.ds(half, half)], sem1)

    zeros = jnp.zeros((LANES,), jnp.float32)

    @plsc.parallel_loop(0, NUM_LEVELS // LANES, unroll=8)
    def _(k):
        hist_v[pl.ds(k * LANES, LANES)] = zeros

    copy0.wait()
    for r in range(rows_per_w):
        if r == half:
            copy1.wait()
        @plsc.parallel_loop(1, chunks, unroll=16,
                            carry=rows_v[r, pl.ds(0, LANES)])
        def acc(i, a):
            return jnp.maximum(a, rows_v[r, pl.ds(i * LANES, LANES)])
        mx = jnp.max(acc)

        @plsc.parallel_loop(0, chunks, unroll=16)
        def _(i):
            x = rows_v[r, pl.ds(i * LANES, LANES)]
            y = x / mx * 255.0
            t = plsc.bitcast(y + 8388608.0, jnp.int32)
            ix = t - 0x4B000000
            ix = jnp.minimum(jnp.maximum(ix, 0), NUM_LEVELS - 1)
            cnt, last = plsc.scan_count(ix)
            plsc.addupdate_scatter(
                hist_v, [ix], cnt.astype(jnp.float32), mask=last)

        @plsc.parallel_loop(0, NUM_LEVELS // LANES, unroll=4)
        def _(j):
            cnt_v[r, pl.ds(j * LANES, LANES)] = hist_v[pl.ds(j * LANES, LANES)]
            hist_v[pl.ds(j * LANES, LANES)] = zeros

    pltpu.sync_copy(cnt_v, out_hbm.at[pl.ds(base, rows_per_w)])


def _sc_hist(d2):
    n_rows, width = d2.shape
    rows_per_w = n_rows // NUM_WORKERS
    mesh = plsc.VectorSubcoreMesh(
        core_axis_name="c", subcore_axis_name="s",
        num_cores=NUM_CORES, num_subcores=NUM_SUBCORES)
    return pl.kernel(
        _sc_hist_body,
        out_type=jax.ShapeDtypeStruct((n_rows, NUM_LEVELS), jnp.float32),
        mesh=mesh,
        scratch_types=[
            pltpu.VMEM((rows_per_w, width), jnp.float32),
            pltpu.VMEM((NUM_LEVELS,), jnp.float32),
            pltpu.VMEM((rows_per_w, NUM_LEVELS), jnp.float32),
            pltpu.SemaphoreType.DMA,
            pltpu.SemaphoreType.DMA,
        ],
        compiler_params=pltpu.CompilerParams(needs_layout_passes=False),
    )(d2)


def _tc_combine_body(counts_ref, spatial_ref, temporal_ref, out_ref):
    counts = counts_ref[...]
    hi = jnp.floor(counts * (1.0 / 256.0))
    lo = counts - hi * 256.0
    sb = spatial_ref[...].astype(jnp.bfloat16)
    dims = (((1,), (0,)), ((), ()))
    sp_hi = lax.dot_general(hi.astype(jnp.bfloat16), sb, dims,
                            preferred_element_type=jnp.float32)
    sp_lo = lax.dot_general(lo.astype(jnp.bfloat16), sb, dims,
                            preferred_element_type=jnp.float32)
    sp = sp_hi * 256.0 + sp_lo
    tmp = temporal_ref[...]
    n_t = tmp.shape[0]
    n_b = out_ref.shape[0]
    for b in range(n_b):
        hv = jnp.sum(sp[b * n_t:(b + 1) * n_t, :] * tmp, axis=0,
                     keepdims=True)
        out_ref[b:b + 1, :] = jnp.sign(hv)


def _tc_combine(counts, spatial, temporal):
    n_rows = counts.shape[0]
    n_t, dim = temporal.shape
    n_b = n_rows // n_t
    return pl.pallas_call(
        _tc_combine_body,
        grid=(1,),
        out_shape=jax.ShapeDtypeStruct((n_b, dim), jnp.float32),
        in_specs=[
            pl.BlockSpec((n_rows, NUM_LEVELS), lambda i: (0, 0)),
            pl.BlockSpec((NUM_LEVELS, dim), lambda i: (0, 0)),
            pl.BlockSpec((n_t, dim), lambda i: (0, 0)),
        ],
        out_specs=pl.BlockSpec((n_b, dim), lambda i: (0, 0)),
    )(counts, spatial, temporal)


@jax.jit
def kernel(data, spatial, temporal):
    b, t, c, h, w = data.shape
    d2 = data.reshape(b * t, c * h * w)
    counts = _sc_hist(d2)
    return _tc_combine(counts, spatial, temporal)


# speedup vs baseline: 1.0263x; 1.0263x over previous
import functools

import jax
import jax.numpy as jnp
from jax import lax
from jax.experimental import pallas as pl
from jax.experimental.pallas import tpu as pltpu
from jax.experimental.pallas import tpu_sc as plsc

NUM_LEVELS = 256
LANES = 16
NUM_CORES = 2
NUM_SUBCORES = 16
NUM_WORKERS = NUM_CORES * NUM_SUBCORES


def _sc_hist_body(data_hbm, out_hbm, rows_v, hist_v, cnt_v, sem0, sem1):
    n_rows, width = data_hbm.shape
    rows_per_w = n_rows // NUM_WORKERS
    half = rows_per_w // 2
    chunks = width // LANES

    wid = lax.axis_index("s") * NUM_CORES + lax.axis_index("c")
    base = wid * rows_per_w
    copy0 = pltpu.async_copy(
        data_hbm.at[pl.ds(base, half)], rows_v.at[pl.ds(0, half)], sem0)
    copy1 = pltpu.async_copy(
        data_hbm.at[pl.ds(base + half, half)],
        rows_v.at[pl.ds(half, half)], sem1)

    zeros = jnp.zeros((LANES,), jnp.float32)

    @plsc.parallel_loop(0, NUM_LEVELS // LANES, unroll=8)
    def _(k):
        hist_v[pl.ds(k * LANES, LANES)] = zeros

    copy0.wait()
    for r in range(rows_per_w):
        if r == half:
            copy1.wait()
        @plsc.parallel_loop(1, chunks, unroll=8,
                            carry=rows_v[r, pl.ds(0, LANES)])
        def acc(i, a):
            return jnp.maximum(a, rows_v[r, pl.ds(i * LANES, LANES)])
        mx = jnp.max(acc)

        @plsc.parallel_loop(0, chunks, unroll=8)
        def _(i):
            x = rows_v[r, pl.ds(i * LANES, LANES)]
            y = x / mx * 255.0
            t = plsc.bitcast(y + 8388608.0, jnp.int32)
            ix = t - 0x4B000000
            ix = jnp.minimum(jnp.maximum(ix, 0), NUM_LEVELS - 1)
            cnt, last = plsc.scan_count(ix)
            plsc.addupdate_scatter(
                hist_v, [ix], cnt.astype(jnp.float32), mask=last)

        @plsc.parallel_loop(0, NUM_LEVELS // LANES, unroll=4)
        def _(j):
            cnt_v[r, pl.ds(j * LANES, LANES)] = hist_v[pl.ds(j * LANES, LANES)]
            hist_v[pl.ds(j * LANES, LANES)] = zeros

    pltpu.sync_copy(cnt_v, out_hbm.at[pl.ds(base, rows_per_w)])


def _sc_hist(d2):
    n_rows, width = d2.shape
    rows_per_w = n_rows // NUM_WORKERS
    mesh = plsc.VectorSubcoreMesh(
        core_axis_name="c", subcore_axis_name="s",
        num_cores=NUM_CORES, num_subcores=NUM_SUBCORES)
    return pl.kernel(
        _sc_hist_body,
        out_type=jax.ShapeDtypeStruct((n_rows, NUM_LEVELS), jnp.float32),
        mesh=mesh,
        scratch_types=[
            pltpu.VMEM((rows_per_w, width), jnp.float32),
            pltpu.VMEM((NUM_LEVELS,), jnp.float32),
            pltpu.VMEM((rows_per_w, NUM_LEVELS), jnp.float32),
            pltpu.SemaphoreType.DMA,
            pltpu.SemaphoreType.DMA,
        ],
        compiler_params=pltpu.CompilerParams(needs_layout_passes=False),
    )(d2)


def _tc_combine_body(counts_ref, spatial_ref, temporal_ref, out_ref):
    counts = counts_ref[...]
    hi = jnp.floor(counts * (1.0 / 256.0))
    lo = counts - hi * 256.0
    sb = spatial_ref[...].astype(jnp.bfloat16)
    dims = (((1,), (0,)), ((), ()))
    sp_hi = lax.dot_general(hi.astype(jnp.bfloat16), sb, dims,
                            preferred_element_type=jnp.float32)
    sp_lo = lax.dot_general(lo.astype(jnp.bfloat16), sb, dims,
                            preferred_element_type=jnp.float32)
    sp = sp_hi * 256.0 + sp_lo
    tmp = temporal_ref[...]
    n_t = tmp.shape[0]
    n_b = out_ref.shape[0]
    for b in range(n_b):
        hv = jnp.sum(sp[b * n_t:(b + 1) * n_t, :] * tmp, axis=0,
                     keepdims=True)
        out_ref[b:b + 1, :] = jnp.sign(hv)


def _tc_combine(counts, spatial, temporal):
    n_rows = counts.shape[0]
    n_t, dim = temporal.shape
    n_b = n_rows // n_t
    return pl.pallas_call(
        _tc_combine_body,
        grid=(1,),
        out_shape=jax.ShapeDtypeStruct((n_b, dim), jnp.float32),
        in_specs=[
            pl.BlockSpec((n_rows, NUM_LEVELS), lambda i: (0, 0)),
            pl.BlockSpec((NUM_LEVELS, dim), lambda i: (0, 0)),
            pl.BlockSpec((n_t, dim), lambda i: (0, 0)),
        ],
        out_specs=pl.BlockSpec((n_b, dim), lambda i: (0, 0)),
    )(counts, spatial, temporal)


@jax.jit
def kernel(data, spatial, temporal):
    b, t, c, h, w = data.shape
    d2 = data.reshape(b * t, c * h * w)
    counts = _sc_hist(d2)
    return _tc_combine(counts, spatial, temporal)


# speedup vs baseline: 1.0393x; 1.0127x over previous
import jax
import jax.numpy as jnp
from jax import lax
from jax.experimental import pallas as pl
from jax.experimental.pallas import tpu as pltpu
from jax.experimental.pallas import tpu_sc as plsc

NUM_LEVELS = 256
LANES = 16
NUM_CORES = 2
NUM_SUBCORES = 16
NUM_WORKERS = NUM_CORES * NUM_SUBCORES


def _sc_hist_body(data_hbm, out_hbm, rows_v, hist_v, cnt_v, sem0, sem1):
    n_rows, width = data_hbm.shape
    rows_per_w = n_rows // NUM_WORKERS
    half = rows_per_w // 2
    chunks = width // LANES

    wid = lax.axis_index("s") * NUM_CORES + lax.axis_index("c")
    base = wid * rows_per_w
    copy0 = pltpu.async_copy(
        data_hbm.at[pl.ds(base, half)], rows_v.at[pl.ds(0, half)], sem0)
    copy1 = pltpu.async_copy(
        data_hbm.at[pl.ds(base + half, half)],
        rows_v.at[pl.ds(half, half)], sem1)

    zeros = jnp.zeros((LANES,), jnp.float32)

    @plsc.parallel_loop(0, NUM_LEVELS // LANES, unroll=8)
    def _(k):
        hist_v[pl.ds(k * LANES, LANES)] = zeros

    copy0.wait()
    for r in range(rows_per_w):
        if r == half:
            copy1.wait()
        @plsc.parallel_loop(1, chunks, unroll=8,
                            carry=rows_v[r, pl.ds(0, LANES)])
        def acc(i, a):
            return jnp.maximum(a, rows_v[r, pl.ds(i * LANES, LANES)])
        mx = jnp.max(acc)

        @plsc.parallel_loop(0, chunks, unroll=8)
        def _(i):
            x = rows_v[r, pl.ds(i * LANES, LANES)]
            y = x / mx * 255.0
            t = plsc.bitcast(y + 8388608.0, jnp.int32)
            ix = t - 0x4B000000
            ix = jnp.minimum(jnp.maximum(ix, 0), NUM_LEVELS - 1)
            cnt, last = plsc.scan_count(ix)
            plsc.addupdate_scatter(
                hist_v, [ix], cnt.astype(jnp.float32), mask=last)

        @plsc.parallel_loop(0, NUM_LEVELS // LANES, unroll=4)
        def _(j):
            cnt_v[r, pl.ds(j * LANES, LANES)] = hist_v[pl.ds(j * LANES, LANES)]
            hist_v[pl.ds(j * LANES, LANES)] = zeros

    pltpu.sync_copy(cnt_v, out_hbm.at[pl.ds(base, rows_per_w)])


def _sc_hist(d2):
    n_rows, width = d2.shape
    rows_per_w = n_rows // NUM_WORKERS
    mesh = plsc.VectorSubcoreMesh(
        core_axis_name="c", subcore_axis_name="s",
        num_cores=NUM_CORES, num_subcores=NUM_SUBCORES)
    return pl.kernel(
        _sc_hist_body,
        out_type=jax.ShapeDtypeStruct((n_rows, NUM_LEVELS), jnp.float32),
        mesh=mesh,
        scratch_types=[
            pltpu.VMEM((rows_per_w, width), jnp.float32),
            pltpu.VMEM((NUM_LEVELS,), jnp.float32),
            pltpu.VMEM((rows_per_w, NUM_LEVELS), jnp.float32),
            pltpu.SemaphoreType.DMA,
            pltpu.SemaphoreType.DMA,
        ],
        compiler_params=pltpu.CompilerParams(needs_layout_passes=False),
    )(d2)


def _tc_combine_body(counts_ref, spatial_ref, temporal_ref, out_ref):
    counts = counts_ref[...]
    hi = jnp.floor(counts * (1.0 / 256.0))
    lo = counts - hi * 256.0
    sb = spatial_ref[...].astype(jnp.bfloat16)
    dims = (((1,), (0,)), ((), ()))
    sp_hi = lax.dot_general(hi.astype(jnp.bfloat16), sb, dims,
                            preferred_element_type=jnp.float32)
    sp_lo = lax.dot_general(lo.astype(jnp.bfloat16), sb, dims,
                            preferred_element_type=jnp.float32)
    sp = sp_hi * 256.0 + sp_lo
    tmp = temporal_ref[...]
    n_t = tmp.shape[0]
    n_b = out_ref.shape[0]
    for b in range(n_b):
        hv = jnp.sum(sp[b * n_t:(b + 1) * n_t, :] * tmp, axis=0,
                     keepdims=True)
        out_ref[b:b + 1, :] = jnp.sign(hv)


def _tc_combine(counts, spatial, temporal):
    n_rows = counts.shape[0]
    n_t, dim = temporal.shape
    n_b = n_rows // n_t
    return pl.pallas_call(
        _tc_combine_body,
        grid=(1,),
        out_shape=jax.ShapeDtypeStruct((n_b, dim), jnp.float32),
        in_specs=[
            pl.BlockSpec((n_rows, NUM_LEVELS), lambda i: (0, 0)),
            pl.BlockSpec((NUM_LEVELS, dim), lambda i: (0, 0)),
            pl.BlockSpec((n_t, dim), lambda i: (0, 0)),
        ],
        out_specs=pl.BlockSpec((n_b, dim), lambda i: (0, 0)),
    )(counts, spatial, temporal)


@jax.jit
def kernel(data, spatial, temporal):
    b, t, c, h, w = data.shape
    d2 = data.reshape(b * t, c * h * w)
    counts = _sc_hist(d2)
    return _tc_combine(counts, spatial, temporal)
